# baseline (speedup 1.0000x reference)
import functools
import math

import jax
import jax.numpy as jnp
from jax import lax
from jax.experimental import pallas as pl
from jax.experimental.pallas import tpu as pltpu
from jax.experimental.pallas import tpu_sc as plsc

N = 128
NV = N * N * N
NW = 32
CH = 768
NSL = CH // 16
P_PAD = NW * CH
NQ = 128
PBL = 2048


def _green_body(pts_ref, xqm_ref, qm_ref, out_ref):
    ptsT = pts_ref[...]
    xqm = xqm_ref[...]
    qm = qm_ref[...]
    px = ptsT[0:1, :]
    py = ptsT[1:2, :]
    pz = ptsT[2:3, :]
    pnorm2 = px * px + py * py + pz * pz
    pdotx = jnp.dot(xqm, ptsT, preferred_element_type=jnp.float32)
    r2 = pnorm2 - 2.0 * pdotx
    eps = jnp.float32(jnp.finfo(jnp.float32).eps)
    inv_r = jnp.where(r2 == 0.0, 1.0 / eps, lax.rsqrt(jnp.abs(r2)))
    inv_r3 = inv_r * inv_r * inv_r
    r1 = jnp.dot(qm, inv_r, preferred_element_type=jnp.float32)
    r2m = jnp.dot(qm, inv_r3, preferred_element_type=jnp.float32)
    g = r1[0:1, :]
    s0 = r2m[0:1, :]
    gx = r2m[1:2, :] - px * s0
    gy = r2m[2:3, :] - py * s0
    gz = r2m[3:4, :] - pz * s0
    ggn = gx * ptsT[3:4, :] + gy * ptsT[4:5, :] + gz * ptsT[5:6, :]
    m = ptsT[6:7, :]
    out_ref[...] = jnp.concatenate(
        [m * g, m * ggn, jnp.zeros((6, PBL), jnp.float32)], axis=0)


_green = pl.pallas_call(
    _green_body,
    grid=(P_PAD // PBL,),
    in_specs=[
        pl.BlockSpec((8, PBL), lambda i: (0, i)),
        pl.BlockSpec((NQ, 8), lambda i: (0, 0)),
        pl.BlockSpec((8, NQ), lambda i: (0, 0)),
    ],
    out_specs=pl.BlockSpec((8, PBL), lambda i: (0, i)),
    out_shape=jax.ShapeDtypeStruct((8, P_PAD), jnp.float32),
)


_sc_mesh = plsc.VectorSubcoreMesh(core_axis_name="c", subcore_axis_name="s")


@functools.partial(
    pl.kernel,
    out_type=jax.ShapeDtypeStruct((NW, 32), jnp.float32),
    mesh=_sc_mesh,
    scratch_types=[
        pltpu.VMEM((56, CH // 128, 128), jnp.int32),
        pltpu.VMEM((56, CH // 128, 128), jnp.float32),
        pltpu.VMEM((8, CH), jnp.float32),
        pltpu.VMEM((32,), jnp.float32),
        pltpu.SemaphoreType.DMA,
    ],
)
def _sc_fd(flat_hbm, idx_hbm, aux_hbm, out_hbm, idx_v, g_v, aux_v, res_v, sem):
    wid = lax.axis_index("s") * 2 + lax.axis_index("c")
    pltpu.sync_copy(idx_hbm.at[wid], idx_v)
    pltpu.sync_copy(aux_hbm.at[wid], aux_v)
    copies = []
    for r in range(56):
        for k in range(CH // 128):
            copies.append(pltpu.async_copy(
                flat_hbm.at[idx_v.at[r, k]], g_v.at[r, k], sem))
    for c in copies:
        c.wait()

    s_v = aux_v[5, 0:16]
    ein_v = aux_v[6, 0:16]
    eout_v = aux_v[7, 0:16]
    zero = jnp.zeros((16,), jnp.float32)

    def body(i, carry):
        acc1, acc2 = carry
        sl = pl.ds(i * 16, 16)
        kk = i // 8
        sl2 = pl.ds((i % 8) * 16, 16)
        m1 = aux_v[0, sl]
        m2 = aux_v[1, sl]
        m3 = aux_v[2, sl]
        gv = aux_v[3, sl]
        ggn = aux_v[4, sl]
        for b in range(4):
            bi = (2 * b) * 7
            bo = bi + 7
            c_i = g_v[bi + 0, kk, sl2]
            xl_i = g_v[bi + 1, kk, sl2]
            xr_i = g_v[bi + 2, kk, sl2]
            yl_i = g_v[bi + 3, kk, sl2]
            yr_i = g_v[bi + 4, kk, sl2]
            zl_i = g_v[bi + 5, kk, sl2]
            zr_i = g_v[bi + 6, kk, sl2]
            c_o = g_v[bo + 0, kk, sl2]
            xl_o = g_v[bo + 1, kk, sl2]
            xr_o = g_v[bo + 2, kk, sl2]
            yl_o = g_v[bo + 3, kk, sl2]
            yr_o = g_v[bo + 4, kk, sl2]
            zl_o = g_v[bo + 5, kk, sl2]
            zr_o = g_v[bo + 6, kk, sl2]
            nd_i = (jnp.where(m1 > 0, c_i - xl_i, xr_i - c_i) * m1
                    + jnp.where(m2 > 0, c_i - yl_i, yr_i - c_i) * m2
                    + jnp.where(m3 > 0, c_i - zl_i, zr_i - c_i) * m3)
            nd_o = (jnp.where(m1 > 0, xr_o - c_o, c_o - xl_o) * m1
                    + jnp.where(m2 > 0, yr_o - c_o, c_o - yl_o) * m2
                    + jnp.where(m3 > 0, zr_o - c_o, c_o - zl_o) * m3)
            t1 = s_v * (c_i - c_o) + gv
            t2 = ein_v * (nd_i + ggn) - eout_v * nd_o
            acc1 = acc1 + t1 * t1
            acc2 = acc2 + t2 * t2
        return acc1, acc2

    acc1, acc2 = lax.fori_loop(0, NSL, body, (zero, zero))
    res_v[0:16] = acc1
    res_v[16:32] = acc2
    pltpu.sync_copy(res_v, out_hbm.at[wid])


def kernel(output, q, xq, x_idx, y_idx, z_idx, normal_x, normal_y, normal_z,
           points, e_in, e_out, dx, dy, dz, weight, data_norm):
    p = x_idx.shape[0]
    pad = P_PAD - p
    s = jnp.float32(1.0) / data_norm

    def padf(a, v=0.0):
        return jnp.pad(a.astype(jnp.float32), (0, pad), constant_values=v)

    m1 = padf(s * normal_x / dx)
    m2 = padf(s * normal_y / dy)
    m3 = padf(s * normal_z / dz)
    mask = jnp.pad(jnp.ones((p,), jnp.float32), (0, pad))
    ones = jnp.ones((P_PAD,), jnp.float32)
    ptsT = jnp.stack([
        padf(points[:, 0], 50.0), padf(points[:, 1], 50.0),
        padf(points[:, 2], 50.0), padf(normal_x), padf(normal_y),
        padf(normal_z), mask, ones,
    ], axis=0)

    nq_pad = NQ - q.shape[0]
    qs = jnp.pad(q / (4.0 * math.pi * e_in), (0, nq_pad))
    xqp = jnp.pad(xq, ((0, nq_pad), (0, 0)), constant_values=100.0)
    xnorm2 = jnp.sum(xqp * xqp, axis=1)
    zq = jnp.zeros((NQ,), jnp.float32)
    xqm = jnp.stack([xqp[:, 0], xqp[:, 1], xqp[:, 2], zq, zq, zq, zq,
                     -0.5 * xnorm2], axis=1)
    qm = jnp.stack([qs, qs * xqp[:, 0], qs * xqp[:, 1], qs * xqp[:, 2],
                    zq, zq, zq, zq], axis=0)

    green = _green(ptsT, xqm, qm)

    aux = jnp.stack([
        m1, m2, m3, green[0], green[1],
        jnp.full((P_PAD,), s, jnp.float32),
        jnp.full((P_PAD,), e_in, jnp.float32),
        jnp.full((P_PAD,), e_out, jnp.float32),
    ], axis=0)
    aux = aux.reshape(8, NW, CH).transpose(1, 0, 2)

    lin = x_idx * (N * N) + y_idx * N + z_idx
    lin_pad = jnp.pad(lin, (0, pad))
    offs = jnp.array([0, -N * N, N * N, -N, N, -1, 1], jnp.int32)
    vols = jnp.arange(8, dtype=jnp.int32) * NV
    combo = (vols[:, None] + offs[None, :]).reshape(56)
    idx_all = jnp.where(mask[None, :] > 0,
                        lin_pad[None, :] + combo[:, None], 0)
    idx_all = idx_all.reshape(56, NW, CH // 128, 128).transpose(1, 0, 2, 3)

    flat = output.reshape(-1)
    part = _sc_fd(flat, idx_all, aux)
    return weight * jnp.sum(part) / (4.0 * p)
